# Initial kernel scaffold; baseline (speedup 1.0000x reference)
import functools

import jax
import jax.numpy as jnp
from jax import lax
from jax.experimental import pallas as pl
from jax.experimental.pallas import tpu as pltpu
from jax.experimental.pallas import tpu_sc as plsc

N = 1024
E = 32768
EDGE_DIM = 16
L = 5
NN = N * N
NW = 32
P = NN // NW
LANES = 16

F32_MAX = jnp.float32(3.4028235e38)


def _tc_table_body(w_ref, a_ref, o_ref):
    o_ref[...] = lax.dot_general(
        w_ref[...], a_ref[...], (((1,), (1,)), ((), ())),
        preferred_element_type=jnp.float32)


def _build_table(w_pad, edge_attr):
    return pl.pallas_call(
        _tc_table_body,
        out_shape=jax.ShapeDtypeStruct((8, E), jnp.float32),
    )(w_pad, edge_attr)


def _sc_gather_body(tbl_hbm, idx_hbm, out_hbm, tbl_v, idx_v, acc_v):
    wid = lax.axis_index("s") * 2 + lax.axis_index("c")
    base = wid * P

    for l in range(L):
        pltpu.sync_copy(tbl_hbm.at[l], tbl_v)
        pltpu.sync_copy(idx_hbm.at[l, pl.ds(base, P)], idx_v)

        def body(i, _, l=l):
            sl = pl.ds(i * LANES, LANES)
            iv = idx_v[sl]
            g = plsc.load_gather(tbl_v, [iv])
            if l == 0:
                acc_v[sl] = g
            elif l == L - 1:
                s = acc_v[sl] + g
                s = jnp.clip(s, -F32_MAX, F32_MAX)
                acc_v[sl] = jnp.where(s != s, jnp.float32(0.0), s)
            else:
                acc_v[sl] = acc_v[sl] + g
            return _

        lax.fori_loop(0, P // LANES, body, 0)

    pltpu.sync_copy(acc_v, out_hbm.at[pl.ds(base, P)])


@functools.partial(
    pl.kernel,
    mesh=plsc.VectorSubcoreMesh(core_axis_name="c", subcore_axis_name="s"),
    out_type=jax.ShapeDtypeStruct((NN,), jnp.float32),
    scratch_types=[
        pltpu.VMEM((E,), jnp.float32),
        pltpu.VMEM((P,), jnp.int32),
        pltpu.VMEM((P,), jnp.float32),
    ],
)
def _sc_gather(tbl_hbm, idx_hbm, out_hbm, tbl_v, idx_v, acc_v):
    _sc_gather_body(tbl_hbm, idx_hbm, out_hbm, tbl_v, idx_v, acc_v)


def kernel(x, edge_attr, edge_paths, edge_weights):
    del x
    w_pad = jnp.zeros((8, EDGE_DIM), jnp.float32).at[:L].set(
        edge_weights.astype(jnp.float32) / L)
    tbl = _build_table(w_pad, edge_attr)
    idx_t = edge_paths.astype(jnp.int32).reshape(NN, L).T
    out = _sc_gather(tbl, idx_t)
    return out.reshape(N, N)


# speedup vs baseline: 77.4070x; 77.4070x over previous
import functools

import jax
import jax.numpy as jnp
from jax import lax
from jax.experimental import pallas as pl
from jax.experimental.pallas import tpu as pltpu
from jax.experimental.pallas import tpu_sc as plsc

N = 1024
E = 32768
EDGE_DIM = 16
L = 5
NN = N * N
NW = 32
P = NN // NW
LANES = 16

F32_MAX = 3.4028235e38


def _tc_table_body(w_ref, a_ref, o_ref):
    o_ref[...] = lax.dot_general(
        w_ref[...], a_ref[...], (((1,), (1,)), ((), ())),
        preferred_element_type=jnp.float32)


def _build_table(w_pad, edge_attr):
    return pl.pallas_call(
        _tc_table_body,
        out_shape=jax.ShapeDtypeStruct((8, E), jnp.float32),
    )(w_pad, edge_attr)


def _sc_gather_body(tbl_hbm, idx_hbm, out_hbm, tbl_v, idx_v, acc_v):
    wid = lax.axis_index("s") * 2 + lax.axis_index("c")
    base = wid * P

    for l in range(L):
        pltpu.sync_copy(tbl_hbm.at[pl.ds(l * E, E)], tbl_v)
        pltpu.sync_copy(idx_hbm.at[pl.ds(l * NN + base, P)], idx_v)

        def body(i, _, l=l):
            sl = pl.ds(i * LANES, LANES)
            iv = idx_v[sl]
            g = plsc.load_gather(tbl_v, [iv])
            if l == 0:
                acc_v[sl] = g
            elif l == L - 1:
                s = acc_v[sl] + g
                s = jnp.clip(s, -F32_MAX, F32_MAX)
                acc_v[sl] = jnp.where(s != s, 0.0, s)
            else:
                acc_v[sl] = acc_v[sl] + g
            return _

        lax.fori_loop(0, P // LANES, body, 0)

    pltpu.sync_copy(acc_v, out_hbm.at[pl.ds(base, P)])


@functools.partial(
    pl.kernel,
    mesh=plsc.VectorSubcoreMesh(core_axis_name="c", subcore_axis_name="s"),
    out_type=jax.ShapeDtypeStruct((NN,), jnp.float32),
    compiler_params=pltpu.CompilerParams(needs_layout_passes=False),
    scratch_types=[
        pltpu.VMEM((E,), jnp.float32),
        pltpu.VMEM((P,), jnp.int32),
        pltpu.VMEM((P,), jnp.float32),
    ],
)
def _sc_gather(tbl_hbm, idx_hbm, out_hbm, tbl_v, idx_v, acc_v):
    _sc_gather_body(tbl_hbm, idx_hbm, out_hbm, tbl_v, idx_v, acc_v)


def kernel(x, edge_attr, edge_paths, edge_weights):
    del x
    w_pad = jnp.zeros((8, EDGE_DIM), jnp.float32).at[:L].set(
        edge_weights.astype(jnp.float32) / L)
    tbl = _build_table(w_pad, edge_attr).reshape(8 * E)
    idx_t = edge_paths.astype(jnp.int32).reshape(NN, L).T.reshape(L * NN)
    out = _sc_gather(tbl, idx_t)
    return out.reshape(N, N)
